# baseline (speedup 1.0000x reference)
import functools

import jax
import jax.numpy as jnp
from jax import lax
from jax.experimental import pallas as pl
from jax.experimental.pallas import tpu as pltpu
from jax.experimental.pallas import tpu_sc as plsc

N_POINTS = 262144
N_LEVELS = 8
N_FEATS = 2
LOD_RES = (16, 32, 64, 128, 256, 512, 1024, 2048)
HASHMAP_SIZE = 2 ** 19
P1 = 2654435761
P2 = 805459861

_LEVEL_SIZES = tuple(int(min((r + 1) ** 3, HASHMAP_SIZE)) for r in LOD_RES)
_LEVEL_OFFS = []
_acc = 0
for _s in _LEVEL_SIZES:
    _LEVEL_OFFS.append(_acc)
    _acc += _s
_LEVEL_OFFS = tuple(_LEVEL_OFFS)
N_TABLE_ROWS = _acc

NW = 32
PPT = N_POINTS // NW
C = 64
NCHUNK = PPT // C
G = C // 16
N_STAGED = 2
N_STREAMED = N_LEVELS - N_STAGED
CW = C * N_STREAMED * 16
SLEN = 2048
ST_ROWS = _LEVEL_OFFS[N_STAGED]
ST_WORDS = ((ST_ROWS * 2 + 15) // 16) * 16


def _corner_hashes(px, py, pz, res):
    posx = px * float(res)
    posy = py * float(res)
    posz = pz * float(res)
    ix = posx.astype(jnp.int32)
    iy = posy.astype(jnp.int32)
    iz = posz.astype(jnp.int32)
    a0 = ix.astype(jnp.uint32)
    b0 = iy.astype(jnp.uint32) * jnp.uint32(P1)
    c0 = iz.astype(jnp.uint32) * jnp.uint32(P2)
    a1 = a0 + jnp.uint32(1)
    b1 = b0 + jnp.uint32(P1)
    c1 = c0 + jnp.uint32(P2)
    ab = (a0 ^ b0, a0 ^ b1, a1 ^ b0, a1 ^ b1)
    hs = []
    for c in range(8):
        dx, dy, dz = (c >> 2) & 1, (c >> 1) & 1, c & 1
        hs.append(ab[2 * dx + dy] ^ (c1 if dz else c0))
    return hs


def _body(xt, tab, out, xb, idx, dst, ob, st, sems, semo):
    wid = lax.axis_index("s") * 2 + lax.axis_index("c")
    base = wid * PPT

    iota = lax.iota(jnp.int32, 16)

    pltpu.sync_copy(tab.at[pl.ds(0, ST_WORDS)], st)

    def load_x(n, slot):
        start = base + n * C
        for d in range(3):
            pltpu.sync_copy(xt.at[pl.ds(d * N_POINTS + start, C)],
                            xb.at[pl.ds(slot * 3 * C + d * C, C)])

    def gen(slot, pbase):
        @pl.loop(0, G)
        def _gen(g):
            px = jnp.clip(xb[pl.ds(slot * 3 * C + 0 * C + g * 16, 16)], 1e-6, 1.0 - 1e-6)
            py = jnp.clip(xb[pl.ds(slot * 3 * C + 1 * C + g * 16, 16)], 1e-6, 1.0 - 1e-6)
            pz = jnp.clip(xb[pl.ds(slot * 3 * C + 2 * C + g * 16, 16)], 1e-6, 1.0 - 1e-6)
            for lvl in range(N_STAGED, N_LEVELS):
                res = LOD_RES[lvl]
                size = _LEVEL_SIZES[lvl]
                off = _LEVEL_OFFS[lvl]
                hs = _corner_hashes(px, py, pz, res)
                rb = (g * N_STREAMED + lvl - N_STAGED) * 256
                for c in range(8):
                    if size & (size - 1) == 0:
                        hid = hs[c] & jnp.uint32(size - 1)
                    else:
                        hid = hs[c] % jnp.uint32(size)
                    w0 = (hid.astype(jnp.int32) + jnp.int32(off)) * 2
                    idx[pl.ds(pbase + rb + 16 * c, 16)] = w0
                    idx[pl.ds(pbase + rb + 128 + 16 * c, 16)] = w0 + 1

    def stream_args(pbase):
        args = []
        for k in range(CW // SLEN):
            s = pl.ds(pbase + k * SLEN, SLEN)
            args.append((tab.at[idx.at[s]], dst.at[s], sems))
        return args

    def fire(pbase):
        for a in stream_args(pbase):
            pltpu.async_copy(*a)

    def drain(pbase):
        for a in stream_args(pbase):
            pltpu.make_async_copy(*a).wait()

    def out_args(n, p):
        start = base + n * C
        return (ob.at[pl.ds(p * C * 16, C * 16)],
                out.at[pl.ds(start * 16, C * 16)], semo)

    def interp(n, slot, p):
        @pl.loop(0, G)
        def _interp(g):
            px = jnp.clip(xb[pl.ds(slot * 3 * C + 0 * C + g * 16, 16)], 1e-6, 1.0 - 1e-6)
            py = jnp.clip(xb[pl.ds(slot * 3 * C + 1 * C + g * 16, 16)], 1e-6, 1.0 - 1e-6)
            pz = jnp.clip(xb[pl.ds(slot * 3 * C + 2 * C + g * 16, 16)], 1e-6, 1.0 - 1e-6)
            obase = p * C * 16 + g * 256 + iota * 16
            pbase = p * CW
            for lvl in range(N_LEVELS):
                res = LOD_RES[lvl]
                posx = px * float(res)
                posy = py * float(res)
                posz = pz * float(res)
                fx = posx - posx.astype(jnp.int32).astype(jnp.float32)
                fy = posy - posy.astype(jnp.int32).astype(jnp.float32)
                fz = posz - posz.astype(jnp.int32).astype(jnp.float32)
                wx = (1.0 - fx, fx)
                wy = (1.0 - fy, fy)
                wz = (1.0 - fz, fz)
                wxy = (wx[0] * wy[0], wx[0] * wy[1], wx[1] * wy[0], wx[1] * wy[1])
                acc0 = jnp.zeros((16,), jnp.float32)
                acc1 = jnp.zeros((16,), jnp.float32)
                if lvl < N_STAGED:
                    size = _LEVEL_SIZES[lvl]
                    off = _LEVEL_OFFS[lvl]
                    hs = _corner_hashes(px, py, pz, res)
                    for c in range(8):
                        dx, dy, dz = (c >> 2) & 1, (c >> 1) & 1, c & 1
                        w = wxy[2 * dx + dy] * wz[dz]
                        hid = hs[c] % jnp.uint32(size)
                        widx = (hid.astype(jnp.int32) + jnp.int32(off)) * 2
                        f0 = plsc.load_gather(st, [widx])
                        f1 = plsc.load_gather(st, [widx + 1])
                        acc0 = acc0 + f0 * w
                        acc1 = acc1 + f1 * w
                else:
                    rb = pbase + (g * N_STREAMED + lvl - N_STAGED) * 256
                    for c in range(8):
                        dx, dy, dz = (c >> 2) & 1, (c >> 1) & 1, c & 1
                        w = wxy[2 * dx + dy] * wz[dz]
                        f0 = dst[pl.ds(rb + 16 * c, 16)]
                        f1 = dst[pl.ds(rb + 128 + 16 * c, 16)]
                        acc0 = acc0 + f0 * w
                        acc1 = acc1 + f1 * w
                plsc.store_scatter(ob, [obase + (2 * lvl)], acc0)
                plsc.store_scatter(ob, [obase + (2 * lvl + 1)], acc1)

    load_x(0, 0)
    gen(0, 0)
    fire(0)
    load_x(1, 1)
    gen(1, CW)

    @pl.loop(0, NCHUNK)
    def _chunk(n):
        p = n & 1
        pbase = p * CW
        slot = lax.rem(n, 3)

        @pl.when(n + 1 < NCHUNK)
        def _():
            fire((1 - p) * CW)

        drain(pbase)

        @pl.when(n + 2 < NCHUNK)
        def _():
            load_x(n + 2, lax.rem(n + 2, 3))
            gen(lax.rem(n + 2, 3), pbase)

        @pl.when(n >= 2)
        def _():
            pltpu.make_async_copy(*out_args(n - 2, p)).wait()

        interp(n, slot, p)
        pltpu.async_copy(*out_args(n, p))

    pltpu.make_async_copy(*out_args(NCHUNK - 2, 0)).wait()
    pltpu.make_async_copy(*out_args(NCHUNK - 1, 1)).wait()


@functools.cache
def _lotd():
    return pl.kernel(
        _body,
        out_type=jax.ShapeDtypeStruct((N_POINTS * N_LEVELS * N_FEATS,), jnp.float32),
        mesh=plsc.VectorSubcoreMesh(core_axis_name="c", subcore_axis_name="s"),
        compiler_params=pltpu.CompilerParams(needs_layout_passes=False),
        scratch_types=[
            pltpu.VMEM((3 * 3 * C,), jnp.float32),
            pltpu.VMEM((2 * CW,), jnp.int32),
            pltpu.VMEM((2 * CW,), jnp.float32),
            pltpu.VMEM((2 * C * 16,), jnp.float32),
            pltpu.VMEM((ST_WORDS,), jnp.float32),
            pltpu.SemaphoreType.DMA,
            pltpu.SemaphoreType.DMA,
        ],
    )


@jax.jit
def kernel(x, grid):
    xt = x.T.reshape(-1)
    flat = _lotd()(xt, grid)
    return flat.reshape(N_POINTS, N_LEVELS * N_FEATS)


# speedup vs baseline: 1.0727x; 1.0727x over previous
import functools

import jax
import jax.numpy as jnp
from jax import lax
from jax.experimental import pallas as pl
from jax.experimental.pallas import tpu as pltpu
from jax.experimental.pallas import tpu_sc as plsc

N_POINTS = 262144
N_LEVELS = 8
N_FEATS = 2
LOD_RES = (16, 32, 64, 128, 256, 512, 1024, 2048)
HASHMAP_SIZE = 2 ** 19
P1 = 2654435761
P2 = 805459861

_LEVEL_SIZES = tuple(int(min((r + 1) ** 3, HASHMAP_SIZE)) for r in LOD_RES)
_LEVEL_OFFS = []
_acc = 0
for _s in _LEVEL_SIZES:
    _LEVEL_OFFS.append(_acc)
    _acc += _s
_LEVEL_OFFS = tuple(_LEVEL_OFFS)
N_TABLE_ROWS = _acc

NW = 32
PPT = N_POINTS // NW
C = 64
NCHUNK = PPT // C
G = C // 16
N_STAGED = 2
N_STREAMED = N_LEVELS - N_STAGED
CW = C * N_STREAMED * 16
SLEN = 2048
ST_ROWS = _LEVEL_OFFS[N_STAGED]
ST_WORDS = ((ST_ROWS * 2 + 15) // 16) * 16


def _corner_hashes(px, py, pz, res):
    posx = px * float(res)
    posy = py * float(res)
    posz = pz * float(res)
    ix = posx.astype(jnp.int32)
    iy = posy.astype(jnp.int32)
    iz = posz.astype(jnp.int32)
    a0 = ix.astype(jnp.uint32)
    b0 = iy.astype(jnp.uint32) * jnp.uint32(P1)
    c0 = iz.astype(jnp.uint32) * jnp.uint32(P2)
    a1 = a0 + jnp.uint32(1)
    b1 = b0 + jnp.uint32(P1)
    c1 = c0 + jnp.uint32(P2)
    ab = (a0 ^ b0, a0 ^ b1, a1 ^ b0, a1 ^ b1)
    hs = []
    for c in range(8):
        dx, dy, dz = (c >> 2) & 1, (c >> 1) & 1, c & 1
        hs.append(ab[2 * dx + dy] ^ (c1 if dz else c0))
    return hs


def _body(xt, tab, out, xb, idx, dst, ob, st, sems, semo):
    wid = lax.axis_index("s") * 2 + lax.axis_index("c")
    base = wid * PPT

    iota = lax.iota(jnp.int32, 16)

    pltpu.sync_copy(tab.at[pl.ds(0, ST_WORDS)], st)

    def load_x(n, slot):
        start = base + n * C
        for d in range(3):
            pltpu.sync_copy(xt.at[pl.ds(d * N_POINTS + start, C)],
                            xb.at[pl.ds(slot * 3 * C + d * C, C)])

    def gen(slot, pbase):
        @pl.loop(0, G)
        def _gen(g):
            px = jnp.clip(xb[pl.ds(slot * 3 * C + 0 * C + g * 16, 16)], 1e-6, 1.0 - 1e-6)
            py = jnp.clip(xb[pl.ds(slot * 3 * C + 1 * C + g * 16, 16)], 1e-6, 1.0 - 1e-6)
            pz = jnp.clip(xb[pl.ds(slot * 3 * C + 2 * C + g * 16, 16)], 1e-6, 1.0 - 1e-6)
            for lvl in range(N_STAGED, N_LEVELS):
                res = LOD_RES[lvl]
                size = _LEVEL_SIZES[lvl]
                off = _LEVEL_OFFS[lvl]
                hs = _corner_hashes(px, py, pz, res)
                rb = (g * N_STREAMED + lvl - N_STAGED) * 256
                for c in range(8):
                    if size & (size - 1) == 0:
                        hid = hs[c] & jnp.uint32(size - 1)
                    else:
                        hid = hs[c] % jnp.uint32(size)
                    w0 = (hid.astype(jnp.int32) + jnp.int32(off)) * 2
                    idx[pl.ds(pbase + rb + 16 * c, 16)] = w0
                    idx[pl.ds(pbase + rb + 128 + 16 * c, 16)] = w0 + 1

    def stream_args(pbase):
        args = []
        for k in range(CW // SLEN):
            s = pl.ds(pbase + k * SLEN, SLEN)
            args.append((tab.at[idx.at[s]], dst.at[s], sems))
        return args

    def fire(pbase):
        for a in stream_args(pbase):
            pltpu.async_copy(*a)

    def drain(pbase):
        for a in stream_args(pbase):
            pltpu.make_async_copy(*a).wait()

    def out_args(n, p):
        start = base + n * C
        return (ob.at[pl.ds(p * C * 16, C * 16)],
                out.at[pl.ds(start * 16, C * 16)], semo)

    def interp(n, slot, p):
        @pl.loop(0, G)
        def _interp(g):
            px = jnp.clip(xb[pl.ds(slot * 3 * C + 0 * C + g * 16, 16)], 1e-6, 1.0 - 1e-6)
            py = jnp.clip(xb[pl.ds(slot * 3 * C + 1 * C + g * 16, 16)], 1e-6, 1.0 - 1e-6)
            pz = jnp.clip(xb[pl.ds(slot * 3 * C + 2 * C + g * 16, 16)], 1e-6, 1.0 - 1e-6)
            obase = p * C * 16 + g * 256 + iota * 16
            pbase = p * CW
            for lvl in range(N_LEVELS):
                res = LOD_RES[lvl]
                posx = px * float(res)
                posy = py * float(res)
                posz = pz * float(res)
                fx = posx - posx.astype(jnp.int32).astype(jnp.float32)
                fy = posy - posy.astype(jnp.int32).astype(jnp.float32)
                fz = posz - posz.astype(jnp.int32).astype(jnp.float32)
                wx = (1.0 - fx, fx)
                wy = (1.0 - fy, fy)
                wz = (1.0 - fz, fz)
                wxy = (wx[0] * wy[0], wx[0] * wy[1], wx[1] * wy[0], wx[1] * wy[1])
                acc0 = jnp.zeros((16,), jnp.float32)
                acc1 = jnp.zeros((16,), jnp.float32)
                if lvl < N_STAGED:
                    size = _LEVEL_SIZES[lvl]
                    off = _LEVEL_OFFS[lvl]
                    hs = _corner_hashes(px, py, pz, res)
                    for c in range(8):
                        dx, dy, dz = (c >> 2) & 1, (c >> 1) & 1, c & 1
                        w = wxy[2 * dx + dy] * wz[dz]
                        hid = hs[c] % jnp.uint32(size)
                        widx = (hid.astype(jnp.int32) + jnp.int32(off)) * 2
                        f0 = plsc.load_gather(st, [widx])
                        f1 = plsc.load_gather(st, [widx + 1])
                        acc0 = acc0 + f0 * w
                        acc1 = acc1 + f1 * w
                else:
                    rb = pbase + (g * N_STREAMED + lvl - N_STAGED) * 256
                    for c in range(8):
                        dx, dy, dz = (c >> 2) & 1, (c >> 1) & 1, c & 1
                        w = wxy[2 * dx + dy] * wz[dz]
                        f0 = dst[pl.ds(rb + 16 * c, 16)]
                        f1 = dst[pl.ds(rb + 128 + 16 * c, 16)]
                        acc0 = acc0 + f0 * w
                        acc1 = acc1 + f1 * w
                plsc.store_scatter(ob, [obase + (2 * lvl)], acc0)
                plsc.store_scatter(ob, [obase + (2 * lvl + 1)], acc1)

    load_x(0, 0)
    gen(0, 0)
    fire(0)
    load_x(1, 1)
    gen(1, CW)

    @pl.loop(0, NCHUNK)
    def _chunk(n):
        p = n & 1
        pbase = p * CW
        slot = lax.rem(n, 3)
        drain(pbase)

        @pl.when(n + 1 < NCHUNK)
        def _():
            fire((1 - p) * CW)

        @pl.when(n + 2 < NCHUNK)
        def _():
            load_x(n + 2, lax.rem(n + 2, 3))
            gen(lax.rem(n + 2, 3), pbase)

        @pl.when(n >= 2)
        def _():
            pltpu.make_async_copy(*out_args(n - 2, p)).wait()

        interp(n, slot, p)
        pltpu.async_copy(*out_args(n, p))

    pltpu.make_async_copy(*out_args(NCHUNK - 2, 0)).wait()
    pltpu.make_async_copy(*out_args(NCHUNK - 1, 1)).wait()


@functools.cache
def _lotd():
    return pl.kernel(
        _body,
        out_type=jax.ShapeDtypeStruct((N_POINTS * N_LEVELS * N_FEATS,), jnp.float32),
        mesh=plsc.VectorSubcoreMesh(core_axis_name="c", subcore_axis_name="s"),
        compiler_params=pltpu.CompilerParams(needs_layout_passes=False),
        scratch_types=[
            pltpu.VMEM((3 * 3 * C,), jnp.float32),
            pltpu.VMEM((2 * CW,), jnp.int32),
            pltpu.VMEM((2 * CW,), jnp.float32),
            pltpu.VMEM((2 * C * 16,), jnp.float32),
            pltpu.VMEM((ST_WORDS,), jnp.float32),
            pltpu.SemaphoreType.DMA,
            pltpu.SemaphoreType.DMA,
        ],
    )


@jax.jit
def kernel(x, grid):
    xt = x.T.reshape(-1)
    flat = _lotd()(xt, grid)
    return flat.reshape(N_POINTS, N_LEVELS * N_FEATS)
